# Initial kernel scaffold; baseline (speedup 1.0000x reference)
import jax
import jax.numpy as jnp
from jax.experimental import pallas as pl


def kernel(node_features1, node_features2, node_features3, nodes_ori, trans, bias):
    raise NotImplementedError("write your pallas kernel here")


# speedup vs baseline: 5.8950x; 5.8950x over previous
import functools

import jax
import jax.numpy as jnp
from jax import lax
from jax.experimental import pallas as pl
from jax.experimental.pallas import tpu as pltpu
from jax.experimental.pallas import tpu_sc as plsc

B = 4096
F = 128
EPS = 1e-8
NUM_IDS = 1_000_000
NC = 2
NS = 16
L = 16
TBL = NUM_IDS // NS
NCHUNK = B // L
ROWS_PER_TILE = B // (NC * NS)


def _tc_body(nf1_ref, nf2_ref, nf3_ref, tr_ref, b_ref, x_ref):
    tr = tr_ref[...]
    b = b_ref[...]
    f1 = jnp.tanh(jnp.dot(nf1_ref[...], tr, preferred_element_type=jnp.float32) + b)
    f2 = jnp.tanh(jnp.dot(nf2_ref[...], tr, preferred_element_type=jnp.float32) + b)
    f3 = jnp.tanh(jnp.dot(nf3_ref[...], tr, preferred_element_type=jnp.float32) + b)
    d12 = jnp.sum(f1 * f2, axis=1, keepdims=True)
    d13 = jnp.sum(f1 * f3, axis=1, keepdims=True)
    n1 = jnp.sqrt(jnp.sum(f1 * f1, axis=1, keepdims=True))
    n2 = jnp.sqrt(jnp.sum(f2 * f2, axis=1, keepdims=True))
    n3 = jnp.sqrt(jnp.sum(f3 * f3, axis=1, keepdims=True))
    c2 = d12 / jnp.maximum(n1 * n2, EPS)
    c3 = d13 / jnp.maximum(n1 * n3, EPS)
    w = c2 + c3
    x_ref[...] = f1 + (c2 * f2 + c3 * f3) / w


def _dense_transform(nf1, nf2, nf3, trans, bias):
    grid = 8
    blk = B // grid
    row_spec = pl.BlockSpec((blk, F), lambda i: (i, 0))
    return pl.pallas_call(
        _tc_body,
        grid=(grid,),
        in_specs=[
            row_spec,
            row_spec,
            row_spec,
            pl.BlockSpec((F, F), lambda i: (0, 0)),
            pl.BlockSpec((1, F), lambda i: (0, 0)),
        ],
        out_specs=row_spec,
        out_shape=jax.ShapeDtypeStruct((B, F), jnp.float32),
    )(nf1, nf2, nf3, trans, bias.reshape(1, F))


def _sc_fm_gather_body(nodes_hbm, x_hbm, out_hbm,
                       nodes_v, tbl, fm_local, shared, tmp, fm_idx, rows, sem):
    cid = lax.axis_index("c")
    sid = lax.axis_index("s")
    base = sid * TBL
    iota = jnp.arange(L, dtype=jnp.int32)

    pltpu.sync_copy(nodes_hbm, nodes_v)

    def chunk_ids(c):
        ids = nodes_v[pl.ds(c * L, L)]
        lid = ids - base
        msk = (lid >= 0) & (lid < TBL)
        lidc = jnp.where(msk, lid, 0)
        return lidc, msk

    def pass1(i, carry):
        c = (NCHUNK - 1) - i
        lidc, msk = chunk_ids(c)
        val = (c << 16) + jnp.zeros((L,), jnp.int32)
        plsc.store_scatter(tbl, [lidc], val, mask=msk)
        return carry

    lax.fori_loop(0, NCHUNK, pass1, 0)

    def pass2(c, carry):
        lidc, msk = chunk_ids(c)
        g = plsc.load_gather(tbl, [lidc])
        sel = msk & ((g >> 16) == c)
        plsc.addupdate_scatter(tbl, [lidc], jnp.int32(1) << iota, mask=sel)
        return carry

    lax.fori_loop(0, NCHUNK, pass2, 0)

    def pass3(c, carry):
        lidc, msk = chunk_ids(c)
        t = plsc.load_gather(tbl, [lidc])
        cw = t >> 16
        bits = t & 0xFFFF
        low = bits & (-bits)
        ctz = (plsc.bitcast(low.astype(jnp.float32), jnp.int32) >> 23) - 127
        fm = cw * L + ctz
        fm_local[pl.ds(c * L, L)] = jnp.where(msk, fm + 1, 0)
        return carry

    lax.fori_loop(0, NCHUNK, pass3, 0)

    pltpu.sync_copy(fm_local, shared.at[sid])
    plsc.subcore_barrier()

    wid = cid * NS + sid
    g0 = wid * ROWS_PER_TILE
    pltpu.sync_copy(shared.at[:, pl.ds(g0, ROWS_PER_TILE)], tmp)
    for j in range(ROWS_PER_TILE // L):
        red = tmp[0, pl.ds(j * L, L)]
        for t in range(1, NS):
            red = jnp.maximum(red, tmp[t, pl.ds(j * L, L)])
        fm_idx[pl.ds(j * L, L)] = red - 1

    pltpu.async_copy(x_hbm.at[fm_idx], rows, sem).wait()
    pltpu.sync_copy(rows, out_hbm.at[pl.ds(g0, ROWS_PER_TILE)])


def _sc_fm_gather(nodes_ori, x):
    mesh = plsc.VectorSubcoreMesh(core_axis_name="c", subcore_axis_name="s")
    return pl.kernel(
        _sc_fm_gather_body,
        out_type=jax.ShapeDtypeStruct((B, F), jnp.float32),
        mesh=mesh,
        compiler_params=pltpu.CompilerParams(needs_layout_passes=False),
        scratch_types=[
            pltpu.VMEM((B,), jnp.int32),
            pltpu.VMEM((TBL,), jnp.int32),
            pltpu.VMEM((B,), jnp.int32),
            pltpu.VMEM_SHARED((NS, B), jnp.int32),
            pltpu.VMEM((NS, ROWS_PER_TILE), jnp.int32),
            pltpu.VMEM((ROWS_PER_TILE,), jnp.int32),
            pltpu.VMEM((ROWS_PER_TILE, F), jnp.float32),
            pltpu.SemaphoreType.DMA,
        ],
    )(nodes_ori, x)


def kernel(node_features1, node_features2, node_features3, nodes_ori, trans, bias):
    x = _dense_transform(node_features1, node_features2, node_features3, trans, bias)
    return _sc_fm_gather(nodes_ori, x)


# speedup vs baseline: 6.8147x; 1.1560x over previous
import functools

import jax
import jax.numpy as jnp
from jax import lax
from jax.experimental import pallas as pl
from jax.experimental.pallas import tpu as pltpu
from jax.experimental.pallas import tpu_sc as plsc

B = 4096
F = 128
EPS = 1e-8
NUM_IDS = 1_000_000
NC = 2
NS = 16
L = 16
TBL = NUM_IDS // NS
NCHUNK = B // L
ROWS_PER_TILE = B // (NC * NS)


def _tc_body(nf1_ref, nf2_ref, nf3_ref, tr_ref, b_ref, x_ref):
    tr = tr_ref[...]
    b = b_ref[...]
    f1 = jnp.tanh(jnp.dot(nf1_ref[...], tr, preferred_element_type=jnp.float32) + b)
    f2 = jnp.tanh(jnp.dot(nf2_ref[...], tr, preferred_element_type=jnp.float32) + b)
    f3 = jnp.tanh(jnp.dot(nf3_ref[...], tr, preferred_element_type=jnp.float32) + b)
    d12 = jnp.sum(f1 * f2, axis=1, keepdims=True)
    d13 = jnp.sum(f1 * f3, axis=1, keepdims=True)
    n1 = jnp.sqrt(jnp.sum(f1 * f1, axis=1, keepdims=True))
    n2 = jnp.sqrt(jnp.sum(f2 * f2, axis=1, keepdims=True))
    n3 = jnp.sqrt(jnp.sum(f3 * f3, axis=1, keepdims=True))
    c2 = d12 / jnp.maximum(n1 * n2, EPS)
    c3 = d13 / jnp.maximum(n1 * n3, EPS)
    w = c2 + c3
    x_ref[...] = f1 + (c2 * f2 + c3 * f3) / w


def _dense_transform(nf1, nf2, nf3, trans, bias):
    grid = 8
    blk = B // grid
    row_spec = pl.BlockSpec((blk, F), lambda i: (i, 0))
    return pl.pallas_call(
        _tc_body,
        grid=(grid,),
        in_specs=[
            row_spec,
            row_spec,
            row_spec,
            pl.BlockSpec((F, F), lambda i: (0, 0)),
            pl.BlockSpec((1, F), lambda i: (0, 0)),
        ],
        out_specs=row_spec,
        out_shape=jax.ShapeDtypeStruct((B, F), jnp.float32),
    )(nf1, nf2, nf3, trans, bias.reshape(1, F))


def _sc_fm_gather_body(nodes_hbm, x_hbm, out_hbm,
                       nodes_v, tbl, fm_local, shared, tmp, fm_idx, rows, sem):
    cid = lax.axis_index("c")
    sid = lax.axis_index("s")
    base = sid * TBL
    iota = jnp.arange(L, dtype=jnp.int32)

    pltpu.sync_copy(nodes_hbm, nodes_v)

    def chunk_ids(c):
        ids = nodes_v[pl.ds(c * L, L)]
        lid = ids - base
        msk = (lid >= 0) & (lid < TBL)
        lidc = jnp.where(msk, lid, 0)
        return lidc, msk

    UNROLL = 8

    def pass1(i, carry):
        cb = (NCHUNK - UNROLL) - i * UNROLL
        for u in range(UNROLL - 1, -1, -1):
            c = cb + u
            lidc, msk = chunk_ids(c)
            val = (c << 16) + jnp.zeros((L,), jnp.int32)
            plsc.store_scatter(tbl, [lidc], val, mask=msk)
        return carry

    lax.fori_loop(0, NCHUNK // UNROLL, pass1, 0)

    @plsc.parallel_loop(0, NCHUNK, step=1, unroll=UNROLL)
    def pass2(c):
        lidc, msk = chunk_ids(c)
        g = plsc.load_gather(tbl, [lidc])
        sel = msk & ((g >> 16) == c)
        plsc.addupdate_scatter(tbl, [lidc], jnp.int32(1) << iota, mask=sel)

    @plsc.parallel_loop(0, NCHUNK, step=1, unroll=UNROLL)
    def pass3(c):
        lidc, msk = chunk_ids(c)
        t = plsc.load_gather(tbl, [lidc])
        cw = t >> 16
        bits = t & 0xFFFF
        low = bits & (-bits)
        ctz = (plsc.bitcast(low.astype(jnp.float32), jnp.int32) >> 23) - 127
        fm = cw * L + ctz
        fm_local[pl.ds(c * L, L)] = jnp.where(msk, fm + 1, 0)

    pltpu.sync_copy(fm_local, shared.at[sid])
    plsc.subcore_barrier()

    wid = cid * NS + sid
    g0 = wid * ROWS_PER_TILE
    pltpu.sync_copy(shared.at[:, pl.ds(g0, ROWS_PER_TILE)], tmp)
    for j in range(ROWS_PER_TILE // L):
        red = tmp[0, pl.ds(j * L, L)]
        for t in range(1, NS):
            red = jnp.maximum(red, tmp[t, pl.ds(j * L, L)])
        fm_idx[pl.ds(j * L, L)] = red - 1

    pltpu.async_copy(x_hbm.at[fm_idx], rows, sem).wait()
    pltpu.sync_copy(rows, out_hbm.at[pl.ds(g0, ROWS_PER_TILE)])


def _sc_fm_gather(nodes_ori, x):
    mesh = plsc.VectorSubcoreMesh(core_axis_name="c", subcore_axis_name="s")
    return pl.kernel(
        _sc_fm_gather_body,
        out_type=jax.ShapeDtypeStruct((B, F), jnp.float32),
        mesh=mesh,
        compiler_params=pltpu.CompilerParams(needs_layout_passes=False),
        scratch_types=[
            pltpu.VMEM((B,), jnp.int32),
            pltpu.VMEM((TBL,), jnp.int32),
            pltpu.VMEM((B,), jnp.int32),
            pltpu.VMEM_SHARED((NS, B), jnp.int32),
            pltpu.VMEM((NS, ROWS_PER_TILE), jnp.int32),
            pltpu.VMEM((ROWS_PER_TILE,), jnp.int32),
            pltpu.VMEM((ROWS_PER_TILE, F), jnp.float32),
            pltpu.SemaphoreType.DMA,
        ],
    )(nodes_ori, x)


def kernel(node_features1, node_features2, node_features3, nodes_ori, trans, bias):
    x = _dense_transform(node_features1, node_features2, node_features3, trans, bias)
    return _sc_fm_gather(nodes_ori, x)


# speedup vs baseline: 7.2684x; 1.0666x over previous
import functools

import jax
import jax.numpy as jnp
from jax import lax
from jax.experimental import pallas as pl
from jax.experimental.pallas import tpu as pltpu
from jax.experimental.pallas import tpu_sc as plsc

B = 4096
F = 128
EPS = 1e-8
NUM_IDS = 1_000_000
NC = 2
NS = 16
L = 16
TBL = NUM_IDS // NS
NCHUNK = B // L
ROWS_PER_TILE = B // (NC * NS)
UNROLL = 8


def _tc_body(nf1_ref, nf2_ref, nf3_ref, tr_ref, b_ref, x_ref):
    tr = tr_ref[...]
    b = b_ref[...]
    f1 = jnp.tanh(jnp.dot(nf1_ref[...], tr, preferred_element_type=jnp.float32) + b)
    f2 = jnp.tanh(jnp.dot(nf2_ref[...], tr, preferred_element_type=jnp.float32) + b)
    f3 = jnp.tanh(jnp.dot(nf3_ref[...], tr, preferred_element_type=jnp.float32) + b)
    d12 = jnp.sum(f1 * f2, axis=1, keepdims=True)
    d13 = jnp.sum(f1 * f3, axis=1, keepdims=True)
    n1 = jnp.sqrt(jnp.sum(f1 * f1, axis=1, keepdims=True))
    n2 = jnp.sqrt(jnp.sum(f2 * f2, axis=1, keepdims=True))
    n3 = jnp.sqrt(jnp.sum(f3 * f3, axis=1, keepdims=True))
    c2 = d12 / jnp.maximum(n1 * n2, EPS)
    c3 = d13 / jnp.maximum(n1 * n3, EPS)
    w = c2 + c3
    x_ref[...] = f1 + (c2 * f2 + c3 * f3) / w


def _dense_transform(nf1, nf2, nf3, trans, bias):
    grid = 8
    blk = B // grid
    row_spec = pl.BlockSpec((blk, F), lambda i: (i, 0))
    return pl.pallas_call(
        _tc_body,
        grid=(grid,),
        in_specs=[
            row_spec,
            row_spec,
            row_spec,
            pl.BlockSpec((F, F), lambda i: (0, 0)),
            pl.BlockSpec((1, F), lambda i: (0, 0)),
        ],
        out_specs=row_spec,
        out_shape=jax.ShapeDtypeStruct((B, F), jnp.float32),
    )(nf1, nf2, nf3, trans, bias.reshape(1, F))


def _sc_fm_body(nodes_hbm, fm_hbm,
                nodes_v, lidm_v, tbl, fm_local, shared, tmp, fm_idx):
    cid = lax.axis_index("c")
    sid = lax.axis_index("s")
    base = sid * TBL
    iota = jnp.arange(L, dtype=jnp.int32)

    pltpu.sync_copy(nodes_hbm, nodes_v)

    def pass1(i, carry):
        cb = (NCHUNK - UNROLL) - i * UNROLL
        for u in range(UNROLL - 1, -1, -1):
            c = cb + u
            ids = nodes_v[pl.ds(c * L, L)]
            lid = ids - base
            msk = (lid >= 0) & (lid < TBL)
            lidm_v[pl.ds(c * L, L)] = jnp.where(msk, lid, -1)
            val = (c << 16) + jnp.zeros((L,), jnp.int32)
            plsc.store_scatter(tbl, [lid], val, mask=msk)
        return carry

    lax.fori_loop(0, NCHUNK // UNROLL, pass1, 0)

    @plsc.parallel_loop(0, NCHUNK, step=1, unroll=UNROLL)
    def pass2(c):
        lid = lidm_v[pl.ds(c * L, L)]
        msk = lid >= 0
        g = plsc.load_gather(tbl, [lid], mask=msk)
        sel = msk & ((g >> 16) == c)
        plsc.addupdate_scatter(tbl, [lid], jnp.int32(1) << iota, mask=sel)

    @plsc.parallel_loop(0, NCHUNK, step=1, unroll=UNROLL)
    def pass3(c):
        lid = lidm_v[pl.ds(c * L, L)]
        msk = lid >= 0
        t = plsc.load_gather(tbl, [lid], mask=msk)
        cw = t >> 16
        bits = t & 0xFFFF
        low = bits & (-bits)
        ctz = (plsc.bitcast(low.astype(jnp.float32), jnp.int32) >> 23) - 127
        fm = cw * L + ctz
        fm_local[pl.ds(c * L, L)] = jnp.where(msk, fm + 1, 0)

    pltpu.sync_copy(fm_local, shared.at[sid])
    plsc.subcore_barrier()

    wid = cid * NS + sid
    g0 = wid * ROWS_PER_TILE
    pltpu.sync_copy(shared.at[:, pl.ds(g0, ROWS_PER_TILE)], tmp)
    for j in range(ROWS_PER_TILE // L):
        red = tmp[0, pl.ds(j * L, L)]
        for t in range(1, NS):
            red = jnp.maximum(red, tmp[t, pl.ds(j * L, L)])
        fm_idx[pl.ds(j * L, L)] = red - 1
    pltpu.sync_copy(fm_idx, fm_hbm.at[pl.ds(g0, ROWS_PER_TILE)])


def _sc_gather_body(fm_hbm, x_hbm, out_hbm, fm_v, rows, sem):
    cid = lax.axis_index("c")
    sid = lax.axis_index("s")
    wid = cid * NS + sid
    g0 = wid * ROWS_PER_TILE
    pltpu.sync_copy(fm_hbm.at[pl.ds(g0, ROWS_PER_TILE)], fm_v)
    pltpu.async_copy(x_hbm.at[fm_v], rows, sem).wait()
    pltpu.sync_copy(rows, out_hbm.at[pl.ds(g0, ROWS_PER_TILE)])


def _sc_mesh():
    return plsc.VectorSubcoreMesh(core_axis_name="c", subcore_axis_name="s")


def _sc_fm(nodes_ori):
    return pl.kernel(
        _sc_fm_body,
        out_type=jax.ShapeDtypeStruct((B,), jnp.int32),
        mesh=_sc_mesh(),
        compiler_params=pltpu.CompilerParams(needs_layout_passes=False),
        scratch_types=[
            pltpu.VMEM((B,), jnp.int32),
            pltpu.VMEM((B,), jnp.int32),
            pltpu.VMEM((TBL,), jnp.int32),
            pltpu.VMEM((B,), jnp.int32),
            pltpu.VMEM_SHARED((NS, B), jnp.int32),
            pltpu.VMEM((NS, ROWS_PER_TILE), jnp.int32),
            pltpu.VMEM((ROWS_PER_TILE,), jnp.int32),
        ],
    )(nodes_ori)


def _sc_gather(fm, x):
    return pl.kernel(
        _sc_gather_body,
        out_type=jax.ShapeDtypeStruct((B, F), jnp.float32),
        mesh=_sc_mesh(),
        compiler_params=pltpu.CompilerParams(needs_layout_passes=False),
        scratch_types=[
            pltpu.VMEM((ROWS_PER_TILE,), jnp.int32),
            pltpu.VMEM((ROWS_PER_TILE, F), jnp.float32),
            pltpu.SemaphoreType.DMA,
        ],
    )(fm, x)


def kernel(node_features1, node_features2, node_features3, nodes_ori, trans, bias):
    fm = _sc_fm(nodes_ori)
    x = _dense_transform(node_features1, node_features2, node_features3, trans, bias)
    return _sc_gather(fm, x)
